# Initial kernel scaffold; baseline (speedup 1.0000x reference)
import jax
import jax.numpy as jnp
from jax.experimental import pallas as pl


def kernel(h, edge_index, edge_values, W, a):
    raise NotImplementedError("write your pallas kernel here")


# speedup vs baseline: 2.1729x; 2.1729x over previous
import jax
import jax.numpy as jnp
from jax import lax
from jax.experimental import pallas as pl
from jax.experimental.pallas import tpu as pltpu
from jax.experimental.pallas import tpu_sc as plsc

N = 10000
E = 160000
D = 256
DV = D // 16

NC = 2
NS = 16
NT = NC * NS
OWNT = 320
NPASS = 2
OWNP = OWNT // NPASS

C = 4000
NCHT = E // C
NPAIR = NCHT // 2
CAP = 3072
FLUSHTH = CAP - 160
RB = 32

_NEG_BIG = -3.0e38


def _dense_body(h_ref, w_ref, a_ref, wh_ref, t_ref):
    wh = jnp.dot(h_ref[...], w_ref[...], preferred_element_type=jnp.float32)
    wh_ref[...] = wh
    t_ref[...] = jnp.dot(wh, a_ref[...], preferred_element_type=jnp.float32)


def _dense(h, W, acat):
    blk = 400
    grid = N // blk
    return pl.pallas_call(
        _dense_body,
        grid=(grid,),
        in_specs=[
            pl.BlockSpec((blk, D), lambda i: (i, 0)),
            pl.BlockSpec((D, D), lambda i: (0, 0)),
            pl.BlockSpec((D, 2), lambda i: (0, 0)),
        ],
        out_specs=[
            pl.BlockSpec((blk, D), lambda i: (i, 0)),
            pl.BlockSpec((blk, 2), lambda i: (i, 0)),
        ],
        out_shape=[
            jax.ShapeDtypeStruct((N, D), jnp.float32),
            jax.ShapeDtypeStruct((N, 2), jnp.float32),
        ],
    )(h, W, acat)


def _sc_kernel(wh_hbm, t1_hbm, t2_hbm, src_hbm, dst_hbm, out_hbm,
               t1_v, t2_v, sa_v, sb_v, da_v, db_v,
               bv_v, bli_v, bdv_v, rows_v, out_v, den_v,
               sem_sa, sem_sb, sem_da, sem_db, sem_g):
    c = lax.axis_index("c")
    s = lax.axis_index("s")
    g = c * NS + s
    nlo = g * OWNT

    zv = jnp.zeros((16,), jnp.float32)
    zi = jnp.zeros((16,), jnp.int32)

    pltpu.sync_copy(t1_hbm, t1_v)
    pltpu.sync_copy(t2_hbm, t2_v)

    def _mx1(i, m):
        return jnp.maximum(m, t1_v[pl.ds(i * 16, 16)])
    m1 = jnp.max(lax.fori_loop(0, N // 16, _mx1,
                               jnp.full((16,), _NEG_BIG, jnp.float32)))

    def _mx2(i, m):
        return jnp.maximum(m, t2_v[pl.ds(i * 16, 16)])
    m2 = jnp.max(lax.fori_loop(0, N // 16, _mx2,
                               jnp.full((16,), _NEG_BIG, jnp.float32)))
    m_stab = 2.0 * jnp.maximum(m1 + m2, 0.0)

    def _zero_batches(nb16):
        def _zb(b, _):
            bv_v[pl.ds(b * 16, 16)] = zv
            bli_v[pl.ds(b * 16, 16)] = zi
            bdv_v[pl.ds(b * 16, 16)] = zi
            return 0
        lax.fori_loop(0, nb16, _zb, 0)

    def _edge_scan(process_fn, flush_fn):
        pltpu.async_copy(src_hbm.at[0], sa_v, sem_sa)
        pltpu.async_copy(dst_hbm.at[0], da_v, sem_da)

        def _pair(i, ptr):
            ci = 2 * i
            pltpu.async_copy(src_hbm.at[ci + 1], sb_v, sem_sb)
            pltpu.async_copy(dst_hbm.at[ci + 1], db_v, sem_db)
            pltpu.make_async_copy(src_hbm.at[ci], sa_v, sem_sa).wait()
            pltpu.make_async_copy(dst_hbm.at[ci], da_v, sem_da).wait()
            ptr = process_fn(sa_v, da_v, flush_fn, ptr)

            @pl.when(ci + 2 < NCHT)
            def _():
                pltpu.async_copy(src_hbm.at[ci + 2], sa_v, sem_sa)
                pltpu.async_copy(dst_hbm.at[ci + 2], da_v, sem_da)

            pltpu.make_async_copy(src_hbm.at[ci + 1], sb_v, sem_sb).wait()
            pltpu.make_async_copy(dst_hbm.at[ci + 1], db_v, sem_db).wait()
            ptr = process_fn(sb_v, db_v, flush_fn, ptr)
            return ptr
        ptr = lax.fori_loop(0, NPAIR, _pair, jnp.int32(0))
        flush_fn(ptr)

    ii = lax.iota(jnp.int32, 16)
    onehot = jnp.where(ii == 0, 1.0, 0.0).astype(jnp.float32)

    def _zero_den(i, _):
        den_v[pl.ds(i * 16, 16)] = zv
        return 0
    lax.fori_loop(0, (OWNT + 16) // 16, _zero_den, 0)

    def _den_flush(ptr):
        nb = (ptr + 15) // 16

        def _blk(b, _):
            exv = bv_v[pl.ds(b * 16, 16)]
            liv = bli_v[pl.ds(b * 16, 16)]
            for lane in range(16):
                e = exv[lane]
                li = liv[lane]
                den_v[pl.ds(li, 16)] = den_v[pl.ds(li, 16)] + e * onehot
            return 0
        lax.fori_loop(0, nb, _blk, 0)

        def _rz(b, _):
            bv_v[pl.ds(b * 16, 16)] = zv
            bli_v[pl.ds(b * 16, 16)] = zi
            return 0
        lax.fori_loop(0, nb, _rz, 0)

    def _den_process(bufS, bufD, flush_fn, ptr):
        def _outer(o, pt):
            def _inner(kk, pt2):
                k = o * 10 + kk
                sv = bufS[pl.ds(k * 16, 16)]
                dv = bufD[pl.ds(k * 16, 16)]
                x = (plsc.load_gather(t1_v, [sv])
                     + plsc.load_gather(t2_v, [dv]))
                lg = jnp.where(x > 0, x, 0.01 * x) * 2.0
                ex = jnp.exp(lg - m_stab)
                own = (sv >= nlo) & (sv < nlo + OWNT)
                li = jnp.where(own, sv - nlo, 0)
                plsc.store_compressed(bv_v.at[pl.ds(pt2, 16)], ex, mask=own)
                plsc.store_compressed(bli_v.at[pl.ds(pt2, 16)], li, mask=own)
                cnt = plsc.all_reduce_population_count(own)
                return pt2 + cnt[0]
            pt = lax.fori_loop(0, 10, _inner, pt)

            @pl.when(pt > FLUSHTH)
            def _():
                flush_fn(pt)
            return jnp.where(pt > FLUSHTH, 0, pt)
        return lax.fori_loop(0, C // 160, _outer, ptr)

    _zero_batches(CAP // 16)
    _edge_scan(_den_process, _den_flush)

    def _pass(p, _p):
        lo = nlo + p * OWNP

        def _zero_out(i, _):
            out_v[i // DV, pl.ds((i % DV) * 16, 16)] = zv
            return 0
        lax.fori_loop(0, OWNP * DV, _zero_out, 0)

        def _row_flush(ptr):
            nb = (ptr + (RB - 1)) // RB

            def _blk(b, _):
                base = b * RB
                pltpu.async_copy(wh_hbm.at[bdv_v.at[pl.ds(base, RB)]],
                                 rows_v, sem_g).wait()
                for grp in range(RB // 16):
                    attv = bv_v[pl.ds(base + grp * 16, 16)]
                    liv = bli_v[pl.ds(base + grp * 16, 16)]
                    for lane in range(16):
                        a = attv[lane]
                        li = liv[lane]
                        r = grp * 16 + lane

                        def _mad(w, _2, a=a, li=li, r=r):
                            out_v[li, pl.ds(w * 16, 16)] = (
                                out_v[li, pl.ds(w * 16, 16)]
                                + a * rows_v[r, pl.ds(w * 16, 16)])
                            return 0
                        lax.fori_loop(0, DV, _mad, 0)
                return 0
            lax.fori_loop(0, nb, _blk, 0)

            def _rz(b, _):
                bv_v[pl.ds(b * 16, 16)] = zv
                bli_v[pl.ds(b * 16, 16)] = zi
                bdv_v[pl.ds(b * 16, 16)] = zi
                return 0
            lax.fori_loop(0, nb * (RB // 16), _rz, 0)

        def _row_process(bufS, bufD, flush_fn, ptr):
            def _outer(o, pt):
                def _inner(kk, pt2):
                    k = o * 10 + kk
                    sv = bufS[pl.ds(k * 16, 16)]
                    dv = bufD[pl.ds(k * 16, 16)]
                    x = (plsc.load_gather(t1_v, [sv])
                         + plsc.load_gather(t2_v, [dv]))
                    lg = jnp.where(x > 0, x, 0.01 * x) * 2.0
                    ex = jnp.exp(lg - m_stab)
                    own = (sv >= lo) & (sv < lo + OWNP)
                    ld = jnp.where(own, sv - nlo, 0)
                    li = jnp.where(own, sv - lo, 0)
                    dn = plsc.load_gather(den_v, [ld])
                    att = jnp.where(own & (dn > 0), ex / dn, 0.0)
                    plsc.store_compressed(bv_v.at[pl.ds(pt2, 16)], att,
                                          mask=own)
                    plsc.store_compressed(bli_v.at[pl.ds(pt2, 16)], li,
                                          mask=own)
                    plsc.store_compressed(bdv_v.at[pl.ds(pt2, 16)], dv,
                                          mask=own)
                    cnt = plsc.all_reduce_population_count(own)
                    return pt2 + cnt[0]
                pt = lax.fori_loop(0, 10, _inner, pt)

                @pl.when(pt > FLUSHTH)
                def _():
                    flush_fn(pt)
                return jnp.where(pt > FLUSHTH, 0, pt)
            return lax.fori_loop(0, C // 160, _outer, ptr)

        _zero_batches(CAP // 16)
        _edge_scan(_row_process, _row_flush)

        def _relu(i, _):
            out_v[i // DV, pl.ds((i % DV) * 16, 16)] = jnp.maximum(
                out_v[i // DV, pl.ds((i % DV) * 16, 16)], 0.0)
            return 0
        lax.fori_loop(0, OWNP * DV, _relu, 0)
        pltpu.sync_copy(out_v, out_hbm.at[pl.ds(nlo + p * OWNP, OWNP)])
        return 0

    lax.fori_loop(0, NPASS, _pass, 0)


def _sc_call(wh, t1, t2, srcc, dstc):
    mesh = plsc.VectorSubcoreMesh(core_axis_name="c", subcore_axis_name="s")
    f = pl.kernel(
        _sc_kernel,
        mesh=mesh,
        compiler_params=pltpu.CompilerParams(needs_layout_passes=False),
        out_type=jax.ShapeDtypeStruct((NT * OWNT, D), jnp.float32),
        scratch_types=[
            pltpu.VMEM((N,), jnp.float32),
            pltpu.VMEM((N,), jnp.float32),
            pltpu.VMEM((C,), jnp.int32),
            pltpu.VMEM((C,), jnp.int32),
            pltpu.VMEM((C,), jnp.int32),
            pltpu.VMEM((C,), jnp.int32),
            pltpu.VMEM((CAP,), jnp.float32),
            pltpu.VMEM((CAP,), jnp.int32),
            pltpu.VMEM((CAP,), jnp.int32),
            pltpu.VMEM((RB, D), jnp.float32),
            pltpu.VMEM((OWNP, D), jnp.float32),
            pltpu.VMEM((OWNT + 16, ), jnp.float32),
            pltpu.SemaphoreType.DMA,
            pltpu.SemaphoreType.DMA,
            pltpu.SemaphoreType.DMA,
            pltpu.SemaphoreType.DMA,
            pltpu.SemaphoreType.DMA,
        ],
    )
    return f(wh, t1, t2, srcc, dstc)


def kernel(h, edge_index, edge_values, W, a):
    del edge_values
    acat = jnp.concatenate([a[:D], a[D:]], axis=1)
    wh, t = _dense(h, W, acat)
    tt = t.T
    t1 = tt[0]
    t2 = tt[1]
    srcc = edge_index[0].reshape(NCHT, C)
    dstc = edge_index[1].reshape(NCHT, C)
    out = _sc_call(wh, t1, t2, srcc, dstc)
    return out[:N]
